# Initial kernel scaffold; baseline (speedup 1.0000x reference)
import jax
import jax.numpy as jnp
from jax.experimental import pallas as pl


def kernel(x, W_in, b_in, op_W1, op_b1, op_W2, op_b2, operator_logits, temper_embed, route_W1, route_b1, route_W2, route_b2, ln_g, ln_b, task_W, task_b, init_tempers):
    raise NotImplementedError("write your pallas kernel here")


# speedup vs baseline: 2.4540x; 2.4540x over previous
import jax
import jax.numpy as jnp
from jax.experimental import pallas as pl
from jax.experimental.pallas import tpu as pltpu

HIGHEST = jax.lax.Precision.HIGHEST

N_TOK = 8192
D_IN = 1024
H = 256
T = 8
HOPS = 2
BLK = 256
NBLK = N_TOK // BLK


def _dense_body(x_ref, w_in_ref, b_in_ref, w1s_ref, b1s_ref, w2s_ref, b2s_ref,
                emb_ref, rw1_ref, rb1_ref, rw2_ref, rb2_ref, lng_ref, lnb_ref,
                tw_ref, tb_ref, t0_ref, out_ref):
    state = jnp.dot(x_ref[...], w_in_ref[...]) + b_in_ref[...]
    tempers = t0_ref[...]
    done = jnp.zeros((BLK, 1), jnp.bool_)
    for _hop in range(HOPS):
        oh8 = (tempers == jax.lax.broadcasted_iota(jnp.int32, (BLK, T), 1)
               ).astype(jnp.float32)
        emb = jnp.dot(oh8, emb_ref[...], precision=HIGHEST)
        rh = (jnp.dot(state, rw1_ref[0:H, :])
              + jnp.dot(emb, rw1_ref[H:H + 4, :])
              + rb1_ref[...])
        rh = jnp.maximum(rh, 0.0)
        logits = jnp.dot(rh, rw2_ref[...]) + rb2_ref[...]
        mx = jnp.max(logits, axis=1, keepdims=True)
        cand = jnp.where(logits == mx,
                         jax.lax.broadcasted_iota(jnp.int32, (BLK, T + 1), 1),
                         T + 1)
        action = jnp.min(cand, axis=1, keepdims=True)
        act = jnp.logical_not(done) & (action < T)
        new_state = state
        for t in range(T):
            h1 = jnp.maximum(
                jnp.dot(state, w1s_ref[t])
                + b1s_ref[t:t + 1, :], 0.0)
            h2 = jnp.maximum(
                jnp.dot(h1, w2s_ref[t])
                + b2s_ref[t:t + 1, :], 0.0) * 1.01
            m = act & (action == t)
            new_state = jnp.where(m, h2, new_state)
        tempers = jnp.where(act, action, tempers)
        done = done | (action == T)
        state = new_state
    mu = jnp.mean(state, axis=1, keepdims=True)
    var = jnp.mean((state - mu) ** 2, axis=1, keepdims=True)
    normed = (state - mu) / jnp.sqrt(var + 1e-5) * lng_ref[...] + lnb_ref[...]
    out_ref[...] = jnp.dot(normed, tw_ref[...]) + tb_ref[...]


def _whole(shape):
    return pl.BlockSpec(shape, lambda i: tuple(0 for _ in shape))


def kernel(x, W_in, b_in, op_W1, op_b1, op_W2, op_b2, operator_logits,
           temper_embed, route_W1, route_b1, route_W2, route_b2, ln_g, ln_b,
           task_W, task_b, init_tempers):
    oi = jnp.argmax(operator_logits, axis=-1)
    ar = jnp.arange(T)
    w1s = op_W1[ar, oi]
    b1s = op_b1[ar, oi]
    w2s = op_W2[ar, oi]
    b2s = op_b2[ar, oi]
    t0 = init_tempers.astype(jnp.int32).reshape(N_TOK, 1)

    out = pl.pallas_call(
        _dense_body,
        grid=(NBLK,),
        in_specs=[
            pl.BlockSpec((BLK, D_IN), lambda i: (i, 0)),
            _whole((D_IN, H)),
            _whole((1, H)),
            _whole((T, H, H)),
            _whole((T, H)),
            _whole((T, H, H)),
            _whole((T, H)),
            _whole((T, 4)),
            _whole((H + 4, H)),
            _whole((1, H)),
            _whole((H, T + 1)),
            _whole((1, T + 1)),
            _whole((1, H)),
            _whole((1, H)),
            _whole((H, 10)),
            _whole((1, 10)),
            pl.BlockSpec((BLK, 1), lambda i: (i, 0)),
        ],
        out_specs=pl.BlockSpec((BLK, 10), lambda i: (i, 0)),
        out_shape=jax.ShapeDtypeStruct((N_TOK, 10), jnp.float32),
    )(x, W_in, b_in.reshape(1, H), w1s, b1s, w2s, b2s, temper_embed,
      route_W1, route_b1.reshape(1, H), route_W2, route_b2.reshape(1, T + 1),
      ln_g.reshape(1, H), ln_b.reshape(1, H), task_W, task_b.reshape(1, 10),
      t0)
    return out
